# baseline (speedup 1.0000x reference)
import functools

import jax
import jax.numpy as jnp
from jax import lax
from jax.experimental import pallas as pl
from jax.experimental.pallas import tpu as pltpu
from jax.experimental.pallas import tpu_sc as plsc

IN_F = 26
HID = 4
BATCH = 16384
L = 16
NC = 2
NS = 16
NW = NC * NS
ROWS = BATCH // NW
CHUNKS = ROWS // L

_mesh = plsc.VectorSubcoreMesh(core_axis_name="c", subcore_axis_name="s")

_SCRATCH = [
    pltpu.VMEM((ROWS * IN_F,), jnp.float32),
    pltpu.VMEM((IN_F * HID,), jnp.float32),
    pltpu.VMEM((HID,), jnp.float32),
    pltpu.VMEM((HID,), jnp.float32),
    pltpu.VMEM((1,), jnp.float32),
    pltpu.VMEM((L,), jnp.int32),
    pltpu.VMEM((ROWS,), jnp.float32),
]


def _mlp_body(x_hbm, w1_hbm, b1_hbm, w2_hbm, b2_hbm, z16_hbm, out_hbm,
              x_v, w1_v, b1_v, w2_v, b2_v, z16_v, out_v):
    wid = lax.axis_index("s") * NC + lax.axis_index("c")
    base = wid * ROWS

    pltpu.sync_copy(x_hbm.at[pl.ds(base * IN_F, ROWS * IN_F)], x_v)
    pltpu.sync_copy(w1_hbm, w1_v)
    pltpu.sync_copy(b1_hbm, b1_v)
    pltpu.sync_copy(w2_hbm, w2_v)
    pltpu.sync_copy(b2_hbm, b2_v)
    pltpu.sync_copy(z16_hbm, z16_v)

    iota = lax.iota(jnp.int32, L)
    iota_f = iota * IN_F
    zv = z16_v[...]

    def _splat(ref, k):
        return plsc.load_gather(ref, [zv + k])

    def body(c, carry):
        s0 = c * L
        xoff = s0 * IN_F
        acc = [_splat(b1_v, j) for j in range(HID)]
        for i in range(IN_F):
            xv = plsc.load_gather(x_v, [iota_f + (xoff + i)])
            for j in range(HID):
                acc[j] = acc[j] + xv * _splat(w1_v, i * HID + j)
        z = _splat(b2_v, 0)
        for j in range(HID):
            a = jnp.abs(acc[j])
            e = jnp.exp(-2.0 * a)
            th = (1.0 - e) / (1.0 + e) * jnp.sign(acc[j])
            z = z + th * _splat(w2_v, j)
        o = 1.0 / (1.0 + jnp.exp(-z))
        plsc.store_scatter(out_v, [s0 + iota], o)
        return carry

    lax.fori_loop(0, CHUNKS, body, jnp.int32(0))
    pltpu.sync_copy(out_v, out_hbm.at[pl.ds(base, ROWS)])


_mlp_sc = functools.partial(
    pl.kernel,
    out_type=jax.ShapeDtypeStruct((BATCH,), jnp.float32),
    mesh=_mesh,
    compiler_params=pltpu.CompilerParams(needs_layout_passes=False),
    scratch_types=_SCRATCH,
)(_mlp_body)


def kernel(x, W1, b1, W2, b2):
    z16 = jnp.zeros((L,), jnp.int32)
    out = _mlp_sc(x.reshape(-1), W1.reshape(-1), b1, W2.reshape(-1), b2, z16)
    return out.reshape(BATCH, 1)


# speedup vs baseline: 1.1828x; 1.1828x over previous
import functools

import jax
import jax.numpy as jnp
from jax import lax
from jax.experimental import pallas as pl
from jax.experimental.pallas import tpu as pltpu
from jax.experimental.pallas import tpu_sc as plsc

IN_F = 26
HID = 4
BATCH = 16384
L = 16
NC = 2
NS = 16
NW = NC * NS
ROWS = BATCH // NW
JAM = 4
STEPS = ROWS // (L * JAM)
NW1 = IN_F * HID
NW1_PAD = 112

_mesh = plsc.VectorSubcoreMesh(core_axis_name="c", subcore_axis_name="s")

_SCRATCH = [
    pltpu.VMEM((ROWS, IN_F), jnp.float32),
    pltpu.VMEM((NW1_PAD,), jnp.float32),
    pltpu.VMEM((L,), jnp.float32),
    pltpu.VMEM((L,), jnp.float32),
    pltpu.VMEM((L,), jnp.float32),
    pltpu.VMEM((L,), jnp.int32),
    pltpu.VMEM((ROWS,), jnp.float32),
]


def _mlp_body(x_hbm, w1_hbm, b1_hbm, w2_hbm, b2_hbm, z16_hbm, out_hbm,
              x_v, w1_v, b1_v, w2_v, b2_v, z16_v, out_v):
    wid = lax.axis_index("s") * NC + lax.axis_index("c")
    base = wid * ROWS

    pltpu.sync_copy(x_hbm.at[pl.ds(base, ROWS), :], x_v)
    pltpu.sync_copy(w1_hbm, w1_v.at[pl.ds(0, NW1)])
    pltpu.sync_copy(b1_hbm, b1_v.at[pl.ds(0, HID)])
    pltpu.sync_copy(w2_hbm, w2_v.at[pl.ds(0, HID)])
    pltpu.sync_copy(b2_hbm, b2_v.at[pl.ds(0, 1)])
    pltpu.sync_copy(z16_hbm, z16_v)

    iota = lax.iota(jnp.int32, L)
    zv = z16_v[...]

    w1src = [w1_v[pl.ds(t * L, L)] for t in range(NW1_PAD // L)]
    b1src = b1_v[...]
    w2src = w2_v[...]
    b2src = b2_v[...]

    def _bc(vec, k):
        return vec[zv + k]

    def body(step, carry):
        s0 = step * (L * JAM)
        b1s = [_bc(b1src, j) for j in range(HID)]
        rows = [s0 + g * L + iota for g in range(JAM)]
        acc = [[b1s[j] for j in range(HID)] for _ in range(JAM)]
        for i in range(IN_F):
            col = zv + i
            xs = [plsc.load_gather(x_v, [rows[g], col]) for g in range(JAM)]
            for j in range(HID):
                w = _bc(w1src[(i * HID + j) // L], (i * HID + j) % L)
                for g in range(JAM):
                    acc[g][j] = acc[g][j] + xs[g] * w
        w2s = [_bc(w2src, j) for j in range(HID)]
        b2s = _bc(b2src, 0)
        for g in range(JAM):
            z = b2s
            for j in range(HID):
                h = acc[g][j]
                a = jnp.abs(h)
                e = jnp.exp(-2.0 * a)
                th = (1.0 - e) / (1.0 + e) * jnp.sign(h)
                z = z + th * w2s[j]
            o = 1.0 / (1.0 + jnp.exp(-z))
            plsc.store_scatter(out_v, [rows[g]], o)
        return carry

    lax.fori_loop(0, STEPS, body, jnp.int32(0))
    pltpu.sync_copy(out_v, out_hbm.at[pl.ds(base, ROWS)])


_mlp_sc = functools.partial(
    pl.kernel,
    out_type=jax.ShapeDtypeStruct((BATCH,), jnp.float32),
    mesh=_mesh,
    compiler_params=pltpu.CompilerParams(needs_layout_passes=False),
    scratch_types=_SCRATCH,
)(_mlp_body)


def kernel(x, W1, b1, W2, b2):
    z16 = jnp.zeros((L,), jnp.int32)
    out = _mlp_sc(x, W1.reshape(-1), b1, W2.reshape(-1), b2, z16)
    return out.reshape(BATCH, 1)


# speedup vs baseline: 1.8099x; 1.5302x over previous
import functools

import jax
import jax.numpy as jnp
from jax.experimental import pallas as pl

IN_F = 26
HID = 4
BATCH = 16384
BLK = 2048


def _mlp_block(x_ref, w1_ref, b1_ref, w2_ref, b2_ref, o_ref):
    x = x_ref[...]
    w1 = w1_ref[...]
    b1 = b1_ref[...].reshape(1, HID)
    w2 = w2_ref[...]
    b2 = b2_ref[...].reshape(1, 1)
    z1 = jax.lax.dot_general(x, w1, (((1,), (0,)), ((), ())),
                             preferred_element_type=jnp.float32)
    h = jnp.tanh(z1 + b1)
    z2 = jax.lax.dot_general(h, w2, (((1,), (0,)), ((), ())),
                             preferred_element_type=jnp.float32)
    o_ref[...] = jax.nn.sigmoid(z2 + b2)


_mlp_tc = pl.pallas_call(
    _mlp_block,
    grid=(BATCH // BLK,),
    in_specs=[
        pl.BlockSpec((BLK, IN_F), lambda i: (i, 0)),
        pl.BlockSpec((IN_F, HID), lambda i: (0, 0)),
        pl.BlockSpec((HID,), lambda i: (0,)),
        pl.BlockSpec((HID, 1), lambda i: (0, 0)),
        pl.BlockSpec((1,), lambda i: (0,)),
    ],
    out_specs=pl.BlockSpec((BLK, 1), lambda i: (i, 0)),
    out_shape=jax.ShapeDtypeStruct((BATCH, 1), jnp.float32),
)


def kernel(x, W1, b1, W2, b2):
    return _mlp_tc(x, W1, b1, W2, b2)


# speedup vs baseline: 2.7724x; 1.5318x over previous
import jax
import jax.numpy as jnp
from jax.experimental import pallas as pl

IN_F = 26
HID = 4
BATCH = 16384
BLK = 2048


def _mlp_block(x_ref, w1_ref, b1_ref, w2_ref, b2_ref, o_ref):
    x = x_ref[...]
    w1 = w1_ref[...]
    b1 = b1_ref[...].reshape(HID, 1)
    w2 = w2_ref[...]
    b2 = b2_ref[...].reshape(1, 1)
    z1t = jax.lax.dot_general(w1, x, (((0,), (1,)), ((), ())),
                              preferred_element_type=jnp.float32)
    h = jnp.tanh(z1t + b1)
    z2 = jnp.sum(h * w2, axis=0, keepdims=True) + b2
    o_ref[...] = jax.nn.sigmoid(z2)


_mlp_tc = pl.pallas_call(
    _mlp_block,
    grid=(BATCH // BLK,),
    in_specs=[
        pl.BlockSpec((BLK, IN_F), lambda i: (i, 0)),
        pl.BlockSpec((IN_F, HID), lambda i: (0, 0)),
        pl.BlockSpec((HID,), lambda i: (0,)),
        pl.BlockSpec((HID, 1), lambda i: (0, 0)),
        pl.BlockSpec((1,), lambda i: (0,)),
    ],
    out_specs=pl.BlockSpec((1, BLK), lambda i: (0, i)),
    out_shape=jax.ShapeDtypeStruct((1, BATCH), jnp.float32),
)


def kernel(x, W1, b1, W2, b2):
    return _mlp_tc(x, W1, b1, W2, b2).reshape(BATCH, 1)


# speedup vs baseline: 6.9860x; 2.5199x over previous
import jax
import jax.numpy as jnp
from jax.experimental import pallas as pl

IN_F = 26
HID = 4
BATCH = 16384
BLK = 2048


def _mlp_block(xt_ref, w1t_ref, b1_ref, w2t_ref, b2_ref, o_ref):
    xt = xt_ref[...]
    w1t = w1t_ref[...]
    b1 = b1_ref[...].reshape(HID, 1)
    w2t = w2t_ref[...]
    b2 = b2_ref[...].reshape(1, 1)
    z1t = jax.lax.dot_general(w1t, xt, (((1,), (0,)), ((), ())),
                              preferred_element_type=jnp.float32)
    h = jnp.tanh(z1t + b1)
    z2 = jax.lax.dot_general(w2t, h, (((1,), (0,)), ((), ())),
                             preferred_element_type=jnp.float32)
    o_ref[...] = jax.nn.sigmoid(z2 + b2)


_mlp_tc = pl.pallas_call(
    _mlp_block,
    grid=(BATCH // BLK,),
    in_specs=[
        pl.BlockSpec((IN_F, BLK), lambda i: (0, i)),
        pl.BlockSpec((HID, IN_F), lambda i: (0, 0)),
        pl.BlockSpec((HID,), lambda i: (0,)),
        pl.BlockSpec((1, HID), lambda i: (0, 0)),
        pl.BlockSpec((1,), lambda i: (0,)),
    ],
    out_specs=pl.BlockSpec((1, BLK), lambda i: (0, i)),
    out_shape=jax.ShapeDtypeStruct((1, BATCH), jnp.float32),
)


def kernel(x, W1, b1, W2, b2):
    return _mlp_tc(x.T, W1.T, b1, W2.T, b2).reshape(BATCH, 1)


# speedup vs baseline: 15.0339x; 2.1520x over previous
import jax
import jax.numpy as jnp
from jax.experimental import pallas as pl

IN_F = 26
HID = 4
BATCH = 16384
BLK = 16384


def _mlp_block(xt_ref, w1t_ref, b1_ref, w2t_ref, b2_ref, o_ref):
    xt = xt_ref[...]
    w1t = w1t_ref[...]
    b1 = b1_ref[...].reshape(HID, 1)
    w2t = w2t_ref[...]
    b2 = b2_ref[...].reshape(1, 1)
    z1t = jax.lax.dot_general(w1t, xt, (((1,), (0,)), ((), ())),
                              preferred_element_type=jnp.float32)
    h = jnp.tanh(z1t + b1)
    z2 = jax.lax.dot_general(w2t, h, (((1,), (0,)), ((), ())),
                             preferred_element_type=jnp.float32)
    o_ref[...] = jax.nn.sigmoid(z2 + b2)


_mlp_tc = pl.pallas_call(
    _mlp_block,
    grid=(BATCH // BLK,),
    in_specs=[
        pl.BlockSpec((IN_F, BLK), lambda i: (0, i)),
        pl.BlockSpec((HID, IN_F), lambda i: (0, 0)),
        pl.BlockSpec((HID,), lambda i: (0,)),
        pl.BlockSpec((1, HID), lambda i: (0, 0)),
        pl.BlockSpec((1,), lambda i: (0,)),
    ],
    out_specs=pl.BlockSpec((1, BLK), lambda i: (0, i)),
    out_shape=jax.ShapeDtypeStruct((1, BATCH), jnp.float32),
)


def kernel(x, W1, b1, W2, b2):
    return _mlp_tc(x.T, W1.T, b1, W2.T, b2).reshape(BATCH, 1)
